# Initial kernel scaffold; baseline (speedup 1.0000x reference)
import jax
import jax.numpy as jnp
from jax.experimental import pallas as pl


def kernel(x, W, b):
    raise NotImplementedError("write your pallas kernel here")


# speedup vs baseline: 1.1445x; 1.1445x over previous
import functools
import jax
import jax.numpy as jnp
from jax.experimental import pallas as pl
from jax.experimental.pallas import tpu as pltpu

_CHUNK = 1024


def _conv_abs_body(x_ref, w_ref, b_ref, out_ref):
    x = x_ref[0]
    w = w_ref[...]
    s = jnp.sum(x * w, axis=0, keepdims=True) + b_ref[0, 0]
    out_ref[...] = jnp.abs(s)[None]


def _select_body(k, nbits, s_ref, out_ref):
    s = s_ref[...]
    u = jax.lax.bitcast_convert_type(s, jnp.int32)
    bsz = s.shape[0]

    def body(i, t):
        bit = nbits - 1 - i
        cand = t | jnp.left_shift(jnp.int32(1), bit)
        c = jnp.sum((u >= cand).astype(jnp.int32), axis=1, keepdims=True)
        return jnp.where(c >= k, cand, t)

    t = jax.lax.fori_loop(0, nbits, body, jnp.zeros((bsz, 1), jnp.int32))
    tf = jax.lax.bitcast_convert_type(t, jnp.float32)
    gt = u > t
    cnt = jnp.sum(gt.astype(jnp.float32), axis=1, keepdims=True)
    ssum = jnp.sum(jnp.where(gt, s, 0.0), axis=1, keepdims=True)
    out_ref[...] = (ssum + (jnp.float32(k) - cnt) * tf) * jnp.float32(1.0 / k)


def kernel(x, W, b):
    B, C, H, Wd = x.shape
    N = H * Wd
    x3 = x.reshape(B, C, N)
    w = W.reshape(C, 1)
    bb = b.reshape(1, 1)
    n_chunks = N // _CHUNK

    s_abs = pl.pallas_call(
        _conv_abs_body,
        grid=(B, n_chunks),
        in_specs=[
            pl.BlockSpec((1, C, _CHUNK), lambda i, j: (i, 0, j)),
            pl.BlockSpec((C, 1), lambda i, j: (0, 0)),
            pl.BlockSpec(memory_space=pltpu.SMEM),
        ],
        out_specs=pl.BlockSpec(
            (1, 1, _CHUNK), lambda i, j, n=n_chunks: (i * n + j, 0, 0)
        ),
        out_shape=jax.ShapeDtypeStruct((B * n_chunks, 1, _CHUNK), jnp.float32),
    )(x3, w, bb)
    s_abs = s_abs.reshape(B, N)

    k = max(int(N * 0.1), 1)
    out = pl.pallas_call(
        functools.partial(_select_body, k, 31),
        out_shape=jax.ShapeDtypeStruct((B, 1), jnp.float32),
    )(s_abs)
    return out


# speedup vs baseline: 1.9483x; 1.7023x over previous
import functools
import jax
import jax.numpy as jnp
from jax.experimental import pallas as pl
from jax.experimental.pallas import tpu as pltpu

_CHUNK = 16384


def _conv_abs_body(x_ref, w_ref, b_ref, out_ref):
    x = x_ref[0]
    w = w_ref[...]
    s = jnp.sum(x * w, axis=0, keepdims=True) + b_ref[0, 0]
    out_ref[...] = jnp.abs(s)[None]


def _select_body(k, nbits, s_ref, out_ref):
    s = s_ref[...]
    u = jax.lax.bitcast_convert_type(s, jnp.int32)
    bsz = s.shape[0]

    def body(i, t):
        bit = nbits - 1 - i
        cand = t | jnp.left_shift(jnp.int32(1), bit)
        c = jnp.sum((u >= cand).astype(jnp.int32), axis=1, keepdims=True)
        return jnp.where(c >= k, cand, t)

    t = jax.lax.fori_loop(0, nbits, body, jnp.zeros((bsz, 1), jnp.int32))
    tf = jax.lax.bitcast_convert_type(t, jnp.float32)
    gt = u > t
    cnt = jnp.sum(gt.astype(jnp.float32), axis=1, keepdims=True)
    ssum = jnp.sum(jnp.where(gt, s, 0.0), axis=1, keepdims=True)
    out_ref[...] = (ssum + (jnp.float32(k) - cnt) * tf) * jnp.float32(1.0 / k)


def kernel(x, W, b):
    B, C, H, Wd = x.shape
    N = H * Wd
    x3 = x.reshape(B, C, N)
    w = W.reshape(C, 1)
    bb = b.reshape(1, 1)
    n_chunks = N // _CHUNK

    s_abs = pl.pallas_call(
        _conv_abs_body,
        grid=(B, n_chunks),
        in_specs=[
            pl.BlockSpec((1, C, _CHUNK), lambda i, j: (i, 0, j)),
            pl.BlockSpec((C, 1), lambda i, j: (0, 0)),
            pl.BlockSpec(memory_space=pltpu.SMEM),
        ],
        out_specs=pl.BlockSpec(
            (1, 1, _CHUNK), lambda i, j, n=n_chunks: (i * n + j, 0, 0)
        ),
        out_shape=jax.ShapeDtypeStruct((B * n_chunks, 1, _CHUNK), jnp.float32),
    )(x3, w, bb)
    s_abs = s_abs.reshape(B, N)

    k = max(int(N * 0.1), 1)
    out = pl.pallas_call(
        functools.partial(_select_body, k, 31),
        out_shape=jax.ShapeDtypeStruct((B, 1), jnp.float32),
    )(s_abs)
    return out


# speedup vs baseline: 2.6248x; 1.3472x over previous
import functools
import jax
import jax.numpy as jnp
from jax.experimental import pallas as pl
from jax.experimental.pallas import tpu as pltpu

_CHUNK = 8192


def _conv_abs_body(x_ref, w_ref, b_ref, out_ref):
    xb = x_ref[...]
    wbd = w_ref[...]
    s = jax.lax.dot_general(
        wbd, xb, (((1,), (0,)), ((), ())),
        preferred_element_type=jnp.float32,
        precision=jax.lax.Precision.HIGHEST,
    ) + b_ref[0, 0]
    out_ref[...] = jnp.abs(s)


def _select_body(k, nbits, s_ref, out_ref):
    s = s_ref[...]
    u = jax.lax.bitcast_convert_type(s, jnp.int32)
    bsz = s.shape[0]

    def body(i, t):
        bit = nbits - 1 - i
        cand = t | jnp.left_shift(jnp.int32(1), bit)
        c = jnp.sum((u >= cand).astype(jnp.int32), axis=1, keepdims=True)
        return jnp.where(c >= k, cand, t)

    t = jax.lax.fori_loop(0, nbits, body, jnp.zeros((bsz, 1), jnp.int32))
    tf = jax.lax.bitcast_convert_type(t, jnp.float32)
    gt = u > t
    cnt = jnp.sum(gt.astype(jnp.float32), axis=1, keepdims=True)
    ssum = jnp.sum(jnp.where(gt, s, 0.0), axis=1, keepdims=True)
    out_ref[...] = (ssum + (jnp.float32(k) - cnt) * tf) * jnp.float32(1.0 / k)


def kernel(x, W, b):
    B, C, H, Wd = x.shape
    N = H * Wd
    x2 = x.reshape(B * C, N)
    wbd = jnp.kron(jnp.eye(B, dtype=jnp.float32), W)
    bb = b.reshape(1, 1)
    n_chunks = N // _CHUNK

    s_abs = pl.pallas_call(
        _conv_abs_body,
        grid=(n_chunks,),
        in_specs=[
            pl.BlockSpec((B * C, _CHUNK), lambda j: (0, j)),
            pl.BlockSpec((B, B * C), lambda j: (0, 0)),
            pl.BlockSpec(memory_space=pltpu.SMEM),
        ],
        out_specs=pl.BlockSpec((B, _CHUNK), lambda j: (0, j)),
        out_shape=jax.ShapeDtypeStruct((B, N), jnp.float32),
    )(x2, wbd, bb)

    k = max(int(N * 0.1), 1)
    out = pl.pallas_call(
        functools.partial(_select_body, k, 31),
        out_shape=jax.ShapeDtypeStruct((B, 1), jnp.float32),
    )(s_abs)
    return out


# speedup vs baseline: 2.9274x; 1.1153x over previous
import functools
import jax
import jax.numpy as jnp
from jax.experimental import pallas as pl
from jax.experimental.pallas import tpu as pltpu

_CHUNK = 8192


def _conv_abs_body(x_ref, w_ref, b_ref, out_ref):
    xb = x_ref[...]
    wbd = w_ref[...]
    s = jax.lax.dot_general(
        wbd, xb, (((1,), (0,)), ((), ())),
        preferred_element_type=jnp.float32,
    ) + b_ref[0, 0]
    out_ref[...] = jnp.abs(s)


def _select_body(k, nbits, s_ref, out_ref):
    s = s_ref[...]
    u = jax.lax.bitcast_convert_type(s, jnp.int32)
    bsz = s.shape[0]

    def body(i, t):
        bit = nbits - 1 - i
        cand = t | jnp.left_shift(jnp.int32(1), bit)
        c = jnp.sum((u >= cand).astype(jnp.int32), axis=1, keepdims=True)
        return jnp.where(c >= k, cand, t)

    t = jax.lax.fori_loop(0, nbits, body, jnp.zeros((bsz, 1), jnp.int32))
    tf = jax.lax.bitcast_convert_type(t, jnp.float32)
    gt = u > t
    cnt = jnp.sum(gt.astype(jnp.float32), axis=1, keepdims=True)
    ssum = jnp.sum(jnp.where(gt, s, 0.0), axis=1, keepdims=True)
    out_ref[...] = (ssum + (jnp.float32(k) - cnt) * tf) * jnp.float32(1.0 / k)


def kernel(x, W, b):
    B, C, H, Wd = x.shape
    N = H * Wd
    x2 = x.reshape(B * C, N)
    wbd = jnp.kron(jnp.eye(B, dtype=jnp.float32), W)
    bb = b.reshape(1, 1)
    n_chunks = N // _CHUNK

    s_abs = pl.pallas_call(
        _conv_abs_body,
        grid=(n_chunks,),
        in_specs=[
            pl.BlockSpec((B * C, _CHUNK), lambda j: (0, j)),
            pl.BlockSpec((B, B * C), lambda j: (0, 0)),
            pl.BlockSpec(memory_space=pltpu.SMEM),
        ],
        out_specs=pl.BlockSpec((B, _CHUNK), lambda j: (0, j)),
        out_shape=jax.ShapeDtypeStruct((B, N), jnp.float32),
    )(x2, wbd, bb)

    k = max(int(N * 0.1), 1)
    out = pl.pallas_call(
        functools.partial(_select_body, k, 31),
        out_shape=jax.ShapeDtypeStruct((B, 1), jnp.float32),
    )(s_abs)
    return out


# speedup vs baseline: 2.9344x; 1.0024x over previous
import functools
import jax
import jax.numpy as jnp
from jax.experimental import pallas as pl
from jax.experimental.pallas import tpu as pltpu

_CHUNK = 4096


def _conv_abs_body(x_ref, w_ref, b_ref, out_ref):
    xb = x_ref[...]
    wbd = w_ref[...]
    s = jax.lax.dot_general(
        wbd, xb, (((1,), (0,)), ((), ())),
        preferred_element_type=jnp.float32,
    ) + b_ref[0, 0]
    out_ref[...] = jnp.abs(s)


def _select_body(k, nbits, s_ref, out_ref):
    s = s_ref[...]
    u = jax.lax.bitcast_convert_type(s, jnp.int32)
    bsz = s.shape[0]

    def body(i, t):
        bit = nbits - 1 - i
        cand = t | jnp.left_shift(jnp.int32(1), bit)
        c = jnp.sum((u >= cand).astype(jnp.int32), axis=1, keepdims=True)
        return jnp.where(c >= k, cand, t)

    t = jax.lax.fori_loop(0, nbits, body, jnp.zeros((bsz, 1), jnp.int32))
    tf = jax.lax.bitcast_convert_type(t, jnp.float32)
    gt = u > t
    cnt = jnp.sum(gt.astype(jnp.float32), axis=1, keepdims=True)
    ssum = jnp.sum(jnp.where(gt, s, 0.0), axis=1, keepdims=True)
    out_ref[...] = (ssum + (jnp.float32(k) - cnt) * tf) * jnp.float32(1.0 / k)


def kernel(x, W, b):
    B, C, H, Wd = x.shape
    N = H * Wd
    x2 = x.reshape(B * C, N)
    wbd = jnp.kron(jnp.eye(B, dtype=jnp.float32), W)
    bb = b.reshape(1, 1)
    n_chunks = N // _CHUNK

    s_abs = pl.pallas_call(
        _conv_abs_body,
        grid=(n_chunks,),
        in_specs=[
            pl.BlockSpec((B * C, _CHUNK), lambda j: (0, j)),
            pl.BlockSpec((B, B * C), lambda j: (0, 0)),
            pl.BlockSpec(memory_space=pltpu.SMEM),
        ],
        out_specs=pl.BlockSpec((B, _CHUNK), lambda j: (0, j)),
        out_shape=jax.ShapeDtypeStruct((B, N), jnp.float32),
    )(x2, wbd, bb)

    k = max(int(N * 0.1), 1)
    out = pl.pallas_call(
        functools.partial(_select_body, k, 31),
        out_shape=jax.ShapeDtypeStruct((B, 1), jnp.float32),
    )(s_abs)
    return out
